# baseline (speedup 1.0000x reference)
import functools

import jax
import jax.numpy as jnp
from jax.experimental import pallas as pl
from jax.experimental.pallas import tpu as pltpu

BATCH = 128
VOCAB = 100000
BLOCK_W = 12800
NUM_BLOCKS = (VOCAB + BLOCK_W - 1) // BLOCK_W
CHUNK = 128
CHUNKS_PER_BLOCK = BLOCK_W // CHUNK
ROWS_PER_GROUP = 8
NUM_GROUPS = BATCH // ROWS_PER_GROUP

_KEY0 = 928981903
_KEY1 = 3453687069
_KS2 = _KEY0 ^ _KEY1 ^ 0x1BD11BDA


def _u32(x):
    return jnp.uint32(x)


def _rotl(x, d):
    return (x << _u32(d)) | (x >> _u32(32 - d))


def _threefry2x32_zero_hi(x1):
    ks = (_KEY0, _KEY1, _KS2)
    rot0 = (13, 15, 26, 6)
    rot1 = (17, 29, 16, 24)
    x0 = x1 + _u32(_KEY0)
    x1 = _rotl(x1, rot0[0]) ^ x0
    first = True
    for rots, ka, kb, inc in (
        (rot0, 1, 2, 1),
        (rot1, 2, 0, 2),
        (rot0, 0, 1, 3),
        (rot1, 1, 2, 4),
        (rot0, 2, 0, 5),
    ):
        for r in rots:
            if first:
                first = False
                continue
            x0 = x0 + x1
            x1 = _rotl(x1, r)
            x1 = x1 ^ x0
        x0 = x0 + _u32(ks[ka])
        x1 = x1 + _u32((ks[kb] + inc) & 0xFFFFFFFF)
    return x0 ^ x1


_TINY = float(jnp.finfo(jnp.float32).tiny)


def _sample_block(logits_ref, out_ref, best_val, best_idx):
    j = pl.program_id(0)
    cshape = (ROWS_PER_GROUP, CHUNK)

    row_off = jax.lax.broadcasted_iota(jnp.uint32, cshape, 0) * _u32(VOCAB)
    lane = jax.lax.broadcasted_iota(jnp.uint32, cshape, 1)
    base_vec = row_off + lane
    lane_i32 = lane.astype(jnp.int32)

    col_block0 = j * BLOCK_W

    @pl.when(j == 0)
    def _init():
        best_val[...] = jnp.full((BATCH, 1), -jnp.inf, jnp.float32)
        best_idx[...] = jnp.zeros((BATCH, 1), jnp.int32)

    for s in range(NUM_GROUPS):
        group_scalar = _u32(s * ROWS_PER_GROUP * VOCAB) + _u32(_KEY1)

        def body(k, carry, s=s, group_scalar=group_scalar):
            vrun, irun = carry
            col0 = col_block0 + k * CHUNK
            x1 = base_vec + (group_scalar + col0.astype(jnp.uint32))
            bits = _threefry2x32_zero_hi(x1)
            fb = (bits >> _u32(9)) | _u32(0x3F800000)
            f = pltpu.bitcast(fb, jnp.float32) - jnp.float32(1.0)
            g = -jnp.log(-jnp.log(jnp.maximum(f, jnp.float32(_TINY))))
            lch = logits_ref[s * ROWS_PER_GROUP : (s + 1) * ROWS_PER_GROUP,
                             pl.ds(k * CHUNK, CHUNK)]
            col = lane_i32 + col0
            v = jnp.where(col < VOCAB, lch + g, -jnp.inf)
            take = v > vrun
            vrun = jnp.maximum(vrun, v)
            irun = jnp.where(take, col, irun)
            return vrun, irun

        vrun0 = jnp.full(cshape, -jnp.inf, jnp.float32)
        irun0 = jnp.zeros(cshape, jnp.int32)
        vrun, irun = jax.lax.fori_loop(
            0, CHUNKS_PER_BLOCK, body, (vrun0, irun0), unroll=2
        )

        gmax = jnp.max(vrun, axis=1, keepdims=True)
        cand = jnp.where(vrun == gmax, irun, jnp.int32(0x7FFFFFFF))
        gidx = jnp.min(cand, axis=1, keepdims=True)

        rows = slice(s * ROWS_PER_GROUP, (s + 1) * ROWS_PER_GROUP)
        prev_v = best_val[rows, :]
        prev_i = best_idx[rows, :]
        better = gmax > prev_v
        best_val[rows, :] = jnp.where(better, gmax, prev_v)
        best_idx[rows, :] = jnp.where(better, gidx, prev_i)

    @pl.when(j == NUM_BLOCKS - 1)
    def _done():
        out_ref[...] = best_idx[...]


@functools.partial(jax.jit, static_argnames=())
def kernel(logits):
    out = pl.pallas_call(
        _sample_block,
        grid=(NUM_BLOCKS,),
        in_specs=[
            pl.BlockSpec((BATCH, BLOCK_W), lambda j: (0, j)),
        ],
        out_specs=pl.BlockSpec((BATCH, 1), lambda j: (0, 0)),
        out_shape=jax.ShapeDtypeStruct((BATCH, 1), jnp.int32),
        scratch_shapes=[
            pltpu.VMEM((BATCH, 1), jnp.float32),
            pltpu.VMEM((BATCH, 1), jnp.int32),
        ],
        compiler_params=pltpu.CompilerParams(
            dimension_semantics=("arbitrary",),
        ),
    )(logits)
    return out.reshape(BATCH)


# speedup vs baseline: 2.0770x; 2.0770x over previous
import functools

import jax
import jax.numpy as jnp
from jax.experimental import pallas as pl
from jax.experimental.pallas import tpu as pltpu

BATCH = 128
VOCAB = 100000
BLOCK_W = 12800
NUM_BLOCKS = (VOCAB + BLOCK_W - 1) // BLOCK_W
CHUNK = 512
CHUNKS_PER_BLOCK = BLOCK_W // CHUNK
ROWS_PER_GROUP = 8
NUM_GROUPS = BATCH // ROWS_PER_GROUP

_KEY0 = 928981903
_KEY1 = 3453687069
_KS2 = _KEY0 ^ _KEY1 ^ 0x1BD11BDA


def _u32(x):
    return jnp.uint32(x)


def _rotl(x, d):
    return (x << _u32(d)) | (x >> _u32(32 - d))


def _threefry2x32_zero_hi(x1):
    ks = (_KEY0, _KEY1, _KS2)
    rot0 = (13, 15, 26, 6)
    rot1 = (17, 29, 16, 24)
    x0 = x1 + _u32(_KEY0)
    x1 = _rotl(x1, rot0[0]) ^ x0
    first = True
    for rots, ka, kb, inc in (
        (rot0, 1, 2, 1),
        (rot1, 2, 0, 2),
        (rot0, 0, 1, 3),
        (rot1, 1, 2, 4),
        (rot0, 2, 0, 5),
    ):
        for r in rots:
            if first:
                first = False
                continue
            x0 = x0 + x1
            x1 = _rotl(x1, r)
            x1 = x1 ^ x0
        x0 = x0 + _u32(ks[ka])
        x1 = x1 + _u32((ks[kb] + inc) & 0xFFFFFFFF)
    return x0 ^ x1


_TINY = float(jnp.finfo(jnp.float32).tiny)


def _sample_block(logits_ref, out_ref, best_val, best_idx):
    j = pl.program_id(0)
    cshape = (ROWS_PER_GROUP, CHUNK)

    row_off = jax.lax.broadcasted_iota(jnp.uint32, cshape, 0) * _u32(VOCAB)
    lane = jax.lax.broadcasted_iota(jnp.uint32, cshape, 1)
    base_vec = row_off + lane
    lane_i32 = lane.astype(jnp.int32)

    col_block0 = j * BLOCK_W

    @pl.when(j == 0)
    def _init():
        best_val[...] = jnp.full((BATCH, 1), -jnp.inf, jnp.float32)
        best_idx[...] = jnp.zeros((BATCH, 1), jnp.int32)

    for s in range(NUM_GROUPS):
        group_scalar = _u32(s * ROWS_PER_GROUP * VOCAB) + _u32(_KEY1)

        def body(k, carry, s=s, group_scalar=group_scalar):
            vrun, irun = carry
            col0 = col_block0 + k * CHUNK
            x1 = base_vec + (group_scalar + col0.astype(jnp.uint32))
            bits = _threefry2x32_zero_hi(x1)
            fb = (bits >> _u32(9)) | _u32(0x3F800000)
            f = pltpu.bitcast(fb, jnp.float32) - jnp.float32(1.0)
            g = -jnp.log(-jnp.log(jnp.maximum(f, jnp.float32(_TINY))))
            lch = logits_ref[s * ROWS_PER_GROUP : (s + 1) * ROWS_PER_GROUP,
                             pl.ds(k * CHUNK, CHUNK)]
            col = lane_i32 + col0
            v = jnp.where(col < VOCAB, lch + g, -jnp.inf)
            take = v > vrun
            vrun = jnp.maximum(vrun, v)
            irun = jnp.where(take, col, irun)
            return vrun, irun

        vrun0 = jnp.full(cshape, -jnp.inf, jnp.float32)
        irun0 = jnp.zeros(cshape, jnp.int32)
        vrun, irun = jax.lax.fori_loop(
            0, CHUNKS_PER_BLOCK, body, (vrun0, irun0), unroll=2
        )

        gmax = jnp.max(vrun, axis=1, keepdims=True)
        cand = jnp.where(vrun == gmax, irun, jnp.int32(0x7FFFFFFF))
        gidx = jnp.min(cand, axis=1, keepdims=True)

        rows = slice(s * ROWS_PER_GROUP, (s + 1) * ROWS_PER_GROUP)
        prev_v = best_val[rows, :]
        prev_i = best_idx[rows, :]
        better = gmax > prev_v
        best_val[rows, :] = jnp.where(better, gmax, prev_v)
        best_idx[rows, :] = jnp.where(better, gidx, prev_i)

    @pl.when(j == NUM_BLOCKS - 1)
    def _done():
        out_ref[...] = best_idx[...]


@functools.partial(jax.jit, static_argnames=())
def kernel(logits):
    out = pl.pallas_call(
        _sample_block,
        grid=(NUM_BLOCKS,),
        in_specs=[
            pl.BlockSpec((BATCH, BLOCK_W), lambda j: (0, j)),
        ],
        out_specs=pl.BlockSpec((BATCH, 1), lambda j: (0, 0)),
        out_shape=jax.ShapeDtypeStruct((BATCH, 1), jnp.int32),
        scratch_shapes=[
            pltpu.VMEM((BATCH, 1), jnp.float32),
            pltpu.VMEM((BATCH, 1), jnp.int32),
        ],
        compiler_params=pltpu.CompilerParams(
            dimension_semantics=("arbitrary",),
        ),
    )(logits)
    return out.reshape(BATCH)


# speedup vs baseline: 2.1843x; 1.0517x over previous
import functools

import jax
import jax.numpy as jnp
from jax.experimental import pallas as pl
from jax.experimental.pallas import tpu as pltpu

BATCH = 128
VOCAB = 100000
BLOCK_W = 12800
NUM_BLOCKS = (VOCAB + BLOCK_W - 1) // BLOCK_W
CHUNK = 512
CHUNKS_PER_BLOCK = BLOCK_W // CHUNK
ROWS_PER_GROUP = 8
NUM_GROUPS = BATCH // ROWS_PER_GROUP

_KEY0 = 928981903
_KEY1 = 3453687069
_KS2 = _KEY0 ^ _KEY1 ^ 0x1BD11BDA


def _u32(x):
    return jnp.uint32(x)


def _rotl(x, d):
    return (x << _u32(d)) | (x >> _u32(32 - d))


def _threefry2x32_zero_hi(x1):
    ks = (_KEY0, _KEY1, _KS2)
    rot0 = (13, 15, 26, 6)
    rot1 = (17, 29, 16, 24)
    x0 = x1 + _u32(_KEY0)
    x1 = _rotl(x1, rot0[0]) ^ x0
    first = True
    for rots, ka, kb, inc in (
        (rot0, 1, 2, 1),
        (rot1, 2, 0, 2),
        (rot0, 0, 1, 3),
        (rot1, 1, 2, 4),
        (rot0, 2, 0, 5),
    ):
        for r in rots:
            if first:
                first = False
                continue
            x0 = x0 + x1
            x1 = _rotl(x1, r)
            x1 = x1 ^ x0
        x0 = x0 + _u32(ks[ka])
        x1 = x1 + _u32((ks[kb] + inc) & 0xFFFFFFFF)
    return x0 ^ x1


_TINY = float(jnp.finfo(jnp.float32).tiny)


def _sample_block(logits_ref, out_ref, best_val, best_idx):
    j = pl.program_id(0)
    cshape = (ROWS_PER_GROUP, CHUNK)

    row_off = jax.lax.broadcasted_iota(jnp.uint32, cshape, 0) * _u32(VOCAB)
    lane = jax.lax.broadcasted_iota(jnp.uint32, cshape, 1)
    base_vec = row_off + lane
    lane_i32 = lane.astype(jnp.int32)

    col_block0 = j * BLOCK_W

    @pl.when(j == 0)
    def _init():
        best_val[...] = jnp.full((BATCH, 1), -jnp.inf, jnp.float32)
        best_idx[...] = jnp.zeros((BATCH, 1), jnp.int32)

    for s in range(NUM_GROUPS):
        group_scalar = _u32(s * ROWS_PER_GROUP * VOCAB) + _u32(_KEY1)

        def body(k, carry, s=s, group_scalar=group_scalar):
            vrun, irun = carry
            col0 = col_block0 + k * CHUNK
            x1 = base_vec + (group_scalar + col0.astype(jnp.uint32))
            bits = _threefry2x32_zero_hi(x1)
            fb = (bits >> _u32(9)) | _u32(0x3F800000)
            f = pltpu.bitcast(fb, jnp.float32) - jnp.float32(1.0)
            g = -jnp.log(-jnp.log(jnp.maximum(f, jnp.float32(_TINY))))
            lch = logits_ref[s * ROWS_PER_GROUP : (s + 1) * ROWS_PER_GROUP,
                             pl.ds(k * CHUNK, CHUNK)]
            col = lane_i32 + col0
            v = jnp.where(col < VOCAB, lch + g, -jnp.inf)
            take = v > vrun
            vrun = jnp.maximum(vrun, v)
            irun = jnp.where(take, col, irun)
            return vrun, irun

        vrun0 = jnp.full(cshape, -jnp.inf, jnp.float32)
        irun0 = jnp.zeros(cshape, jnp.int32)
        vrun, irun = jax.lax.fori_loop(
            0, CHUNKS_PER_BLOCK, body, (vrun0, irun0), unroll=4
        )

        gmax = jnp.max(vrun, axis=1, keepdims=True)
        cand = jnp.where(vrun == gmax, irun, jnp.int32(0x7FFFFFFF))
        gidx = jnp.min(cand, axis=1, keepdims=True)

        rows = slice(s * ROWS_PER_GROUP, (s + 1) * ROWS_PER_GROUP)
        prev_v = best_val[rows, :]
        prev_i = best_idx[rows, :]
        better = gmax > prev_v
        best_val[rows, :] = jnp.where(better, gmax, prev_v)
        best_idx[rows, :] = jnp.where(better, gidx, prev_i)

    @pl.when(j == NUM_BLOCKS - 1)
    def _done():
        out_ref[...] = best_idx[...]


@functools.partial(jax.jit, static_argnames=())
def kernel(logits):
    out = pl.pallas_call(
        _sample_block,
        grid=(NUM_BLOCKS,),
        in_specs=[
            pl.BlockSpec((BATCH, BLOCK_W), lambda j: (0, j)),
        ],
        out_specs=pl.BlockSpec((BATCH, 1), lambda j: (0, 0)),
        out_shape=jax.ShapeDtypeStruct((BATCH, 1), jnp.int32),
        scratch_shapes=[
            pltpu.VMEM((BATCH, 1), jnp.float32),
            pltpu.VMEM((BATCH, 1), jnp.int32),
        ],
        compiler_params=pltpu.CompilerParams(
            dimension_semantics=("arbitrary",),
        ),
    )(logits)
    return out.reshape(BATCH)


# speedup vs baseline: 2.2563x; 1.0329x over previous
import functools

import jax
import jax.numpy as jnp
from jax.experimental import pallas as pl
from jax.experimental.pallas import tpu as pltpu

BATCH = 128
VOCAB = 100000
BLOCK_W = 12800
NUM_BLOCKS = (VOCAB + BLOCK_W - 1) // BLOCK_W
CHUNK = 1280
CHUNKS_PER_BLOCK = BLOCK_W // CHUNK
ROWS_PER_GROUP = 8
NUM_GROUPS = BATCH // ROWS_PER_GROUP

_KEY0 = 928981903
_KEY1 = 3453687069
_KS2 = _KEY0 ^ _KEY1 ^ 0x1BD11BDA


def _u32(x):
    return jnp.uint32(x)


def _rotl(x, d):
    return (x << _u32(d)) | (x >> _u32(32 - d))


def _threefry2x32_zero_hi(x1):
    ks = (_KEY0, _KEY1, _KS2)
    rot0 = (13, 15, 26, 6)
    rot1 = (17, 29, 16, 24)
    x0 = x1 + _u32(_KEY0)
    x1 = _rotl(x1, rot0[0]) ^ x0
    first = True
    for rots, ka, kb, inc in (
        (rot0, 1, 2, 1),
        (rot1, 2, 0, 2),
        (rot0, 0, 1, 3),
        (rot1, 1, 2, 4),
        (rot0, 2, 0, 5),
    ):
        for r in rots:
            if first:
                first = False
                continue
            x0 = x0 + x1
            x1 = _rotl(x1, r)
            x1 = x1 ^ x0
        x0 = x0 + _u32(ks[ka])
        x1 = x1 + _u32((ks[kb] + inc) & 0xFFFFFFFF)
    return x0 ^ x1


_TINY = float(jnp.finfo(jnp.float32).tiny)


def _sample_block(logits_ref, out_ref, best_val, best_idx):
    j = pl.program_id(0)
    cshape = (ROWS_PER_GROUP, CHUNK)

    row_off = jax.lax.broadcasted_iota(jnp.uint32, cshape, 0) * _u32(VOCAB)
    lane = jax.lax.broadcasted_iota(jnp.uint32, cshape, 1)
    base_vec = row_off + lane
    lane_i32 = lane.astype(jnp.int32)

    col_block0 = j * BLOCK_W

    @pl.when(j == 0)
    def _init():
        best_val[...] = jnp.full((BATCH, 1), -jnp.inf, jnp.float32)
        best_idx[...] = jnp.zeros((BATCH, 1), jnp.int32)

    for s in range(NUM_GROUPS):
        group_scalar = _u32(s * ROWS_PER_GROUP * VOCAB) + _u32(_KEY1)

        def body(k, carry, s=s, group_scalar=group_scalar):
            vrun, irun = carry
            col0 = col_block0 + k * CHUNK
            x1 = base_vec + (group_scalar + col0.astype(jnp.uint32))
            bits = _threefry2x32_zero_hi(x1)
            fb = (bits >> _u32(9)) | _u32(0x3F800000)
            f = pltpu.bitcast(fb, jnp.float32) - jnp.float32(1.0)
            g = -jnp.log(-jnp.log(jnp.maximum(f, jnp.float32(_TINY))))
            lch = logits_ref[s * ROWS_PER_GROUP : (s + 1) * ROWS_PER_GROUP,
                             pl.ds(k * CHUNK, CHUNK)]
            col = lane_i32 + col0
            v = jnp.where(col < VOCAB, lch + g, -jnp.inf)
            take = v > vrun
            vrun = jnp.maximum(vrun, v)
            irun = jnp.where(take, col, irun)
            return vrun, irun

        vrun0 = jnp.full(cshape, -jnp.inf, jnp.float32)
        irun0 = jnp.zeros(cshape, jnp.int32)
        vrun, irun = jax.lax.fori_loop(
            0, CHUNKS_PER_BLOCK, body, (vrun0, irun0), unroll=2
        )

        gmax = jnp.max(vrun, axis=1, keepdims=True)
        cand = jnp.where(vrun == gmax, irun, jnp.int32(0x7FFFFFFF))
        gidx = jnp.min(cand, axis=1, keepdims=True)

        rows = slice(s * ROWS_PER_GROUP, (s + 1) * ROWS_PER_GROUP)
        prev_v = best_val[rows, :]
        prev_i = best_idx[rows, :]
        better = gmax > prev_v
        best_val[rows, :] = jnp.where(better, gmax, prev_v)
        best_idx[rows, :] = jnp.where(better, gidx, prev_i)

    @pl.when(j == NUM_BLOCKS - 1)
    def _done():
        out_ref[...] = best_idx[...]


@functools.partial(jax.jit, static_argnames=())
def kernel(logits):
    out = pl.pallas_call(
        _sample_block,
        grid=(NUM_BLOCKS,),
        in_specs=[
            pl.BlockSpec((BATCH, BLOCK_W), lambda j: (0, j)),
        ],
        out_specs=pl.BlockSpec((BATCH, 1), lambda j: (0, 0)),
        out_shape=jax.ShapeDtypeStruct((BATCH, 1), jnp.int32),
        scratch_shapes=[
            pltpu.VMEM((BATCH, 1), jnp.float32),
            pltpu.VMEM((BATCH, 1), jnp.int32),
        ],
        compiler_params=pltpu.CompilerParams(
            dimension_semantics=("arbitrary",),
        ),
    )(logits)
    return out.reshape(BATCH)


# speedup vs baseline: 2.3267x; 1.0312x over previous
import functools

import jax
import jax.numpy as jnp
from jax.experimental import pallas as pl
from jax.experimental.pallas import tpu as pltpu

BATCH = 128
VOCAB = 100000
BLOCK_W = 12800
NUM_BLOCKS = (VOCAB + BLOCK_W - 1) // BLOCK_W
CHUNK = 256
CHUNKS_PER_BLOCK = BLOCK_W // CHUNK
ROWS_PER_GROUP = 8
NUM_GROUPS = BATCH // ROWS_PER_GROUP

_KEY0 = 928981903
_KEY1 = 3453687069
_KS2 = _KEY0 ^ _KEY1 ^ 0x1BD11BDA


def _u32(x):
    return jnp.uint32(x)


def _rotl(x, d):
    return (x << _u32(d)) | (x >> _u32(32 - d))


def _threefry2x32_zero_hi(x1):
    ks = (_KEY0, _KEY1, _KS2)
    rot0 = (13, 15, 26, 6)
    rot1 = (17, 29, 16, 24)
    x0 = x1 + _u32(_KEY0)
    x1 = _rotl(x1, rot0[0]) ^ x0
    first = True
    for rots, ka, kb, inc in (
        (rot0, 1, 2, 1),
        (rot1, 2, 0, 2),
        (rot0, 0, 1, 3),
        (rot1, 1, 2, 4),
        (rot0, 2, 0, 5),
    ):
        for r in rots:
            if first:
                first = False
                continue
            x0 = x0 + x1
            x1 = _rotl(x1, r)
            x1 = x1 ^ x0
        x0 = x0 + _u32(ks[ka])
        x1 = x1 + _u32((ks[kb] + inc) & 0xFFFFFFFF)
    return x0 ^ x1


_TINY = float(jnp.finfo(jnp.float32).tiny)


def _sample_block(logits_ref, out_ref, best_val, best_idx):
    s = pl.program_id(0)
    j = pl.program_id(1)
    cshape = (ROWS_PER_GROUP, CHUNK)

    row_off = jax.lax.broadcasted_iota(jnp.uint32, cshape, 0) * _u32(VOCAB)
    lane = jax.lax.broadcasted_iota(jnp.uint32, cshape, 1)
    base_vec = row_off + lane
    lane_i32 = lane.astype(jnp.int32)

    col_block0 = j * BLOCK_W
    group_scalar = (
        s.astype(jnp.uint32) * _u32(ROWS_PER_GROUP * VOCAB)
        + _u32(_KEY1)
        + col_block0.astype(jnp.uint32)
    )

    vrun = jnp.full(cshape, -jnp.inf, jnp.float32)
    irun = jnp.zeros(cshape, jnp.int32)

    for k in range(CHUNKS_PER_BLOCK):
        x1 = base_vec + (group_scalar + _u32(k * CHUNK))
        bits = _threefry2x32_zero_hi(x1)
        fb = (bits >> _u32(9)) | _u32(0x3F800000)
        f = pltpu.bitcast(fb, jnp.float32) - jnp.float32(1.0)
        g = -jnp.log(-jnp.log(jnp.maximum(f, jnp.float32(_TINY))))
        lch = logits_ref[:, k * CHUNK : (k + 1) * CHUNK]
        col = lane_i32 + (col_block0 + k * CHUNK)
        v = jnp.where(col < VOCAB, lch + g, -jnp.inf)
        take = v > vrun
        vrun = jnp.maximum(vrun, v)
        irun = jnp.where(take, col, irun)

    gmax = jnp.max(vrun, axis=1, keepdims=True)
    cand = jnp.where(vrun == gmax, irun, jnp.int32(0x7FFFFFFF))
    gidx = jnp.min(cand, axis=1, keepdims=True)

    @pl.when(j == 0)
    def _init():
        best_val[...] = jnp.full((ROWS_PER_GROUP, 1), -jnp.inf, jnp.float32)
        best_idx[...] = jnp.zeros((ROWS_PER_GROUP, 1), jnp.int32)

    prev_v = best_val[...]
    prev_i = best_idx[...]
    better = gmax > prev_v
    best_val[...] = jnp.where(better, gmax, prev_v)
    best_idx[...] = jnp.where(better, gidx, prev_i)

    @pl.when(j == NUM_BLOCKS - 1)
    def _done():
        out_ref[...] = best_idx[...]


@functools.partial(jax.jit, static_argnames=())
def kernel(logits):
    out = pl.pallas_call(
        _sample_block,
        grid=(NUM_GROUPS, NUM_BLOCKS),
        in_specs=[
            pl.BlockSpec((ROWS_PER_GROUP, BLOCK_W), lambda s, j: (s, j)),
        ],
        out_specs=pl.BlockSpec((ROWS_PER_GROUP, 1), lambda s, j: (s, 0)),
        out_shape=jax.ShapeDtypeStruct((BATCH, 1), jnp.int32),
        scratch_shapes=[
            pltpu.VMEM((ROWS_PER_GROUP, 1), jnp.float32),
            pltpu.VMEM((ROWS_PER_GROUP, 1), jnp.int32),
        ],
        compiler_params=pltpu.CompilerParams(
            dimension_semantics=("arbitrary", "arbitrary"),
        ),
    )(logits)
    return out.reshape(BATCH)


# speedup vs baseline: 2.4405x; 1.0489x over previous
import functools

import jax
import jax.numpy as jnp
from jax.experimental import pallas as pl
from jax.experimental.pallas import tpu as pltpu

BATCH = 128
VOCAB = 100000
BLOCK_W = 25600
NUM_BLOCKS = (VOCAB + BLOCK_W - 1) // BLOCK_W
CHUNK = 256
CHUNKS_PER_BLOCK = BLOCK_W // CHUNK
ROWS_PER_GROUP = 8
NUM_GROUPS = BATCH // ROWS_PER_GROUP

_KEY0 = 928981903
_KEY1 = 3453687069
_KS2 = _KEY0 ^ _KEY1 ^ 0x1BD11BDA


def _u32(x):
    return jnp.uint32(x)


def _rotl(x, d):
    return (x << _u32(d)) | (x >> _u32(32 - d))


def _threefry2x32_zero_hi(x1):
    ks = (_KEY0, _KEY1, _KS2)
    rot0 = (13, 15, 26, 6)
    rot1 = (17, 29, 16, 24)
    x0 = x1 + _u32(_KEY0)
    x1 = _rotl(x1, rot0[0]) ^ x0
    first = True
    for rots, ka, kb, inc in (
        (rot0, 1, 2, 1),
        (rot1, 2, 0, 2),
        (rot0, 0, 1, 3),
        (rot1, 1, 2, 4),
        (rot0, 2, 0, 5),
    ):
        for r in rots:
            if first:
                first = False
                continue
            x0 = x0 + x1
            x1 = _rotl(x1, r)
            x1 = x1 ^ x0
        x0 = x0 + _u32(ks[ka])
        x1 = x1 + _u32((ks[kb] + inc) & 0xFFFFFFFF)
    return x0 ^ x1


_TINY = float(jnp.finfo(jnp.float32).tiny)


def _sample_block(logits_ref, out_ref, best_val, best_idx):
    s = pl.program_id(0)
    j = pl.program_id(1)
    cshape = (ROWS_PER_GROUP, CHUNK)

    row_off = jax.lax.broadcasted_iota(jnp.uint32, cshape, 0) * _u32(VOCAB)
    lane = jax.lax.broadcasted_iota(jnp.uint32, cshape, 1)
    base_vec = row_off + lane
    lane_i32 = lane.astype(jnp.int32)

    col_block0 = j * BLOCK_W
    group_scalar = (
        s.astype(jnp.uint32) * _u32(ROWS_PER_GROUP * VOCAB)
        + _u32(_KEY1)
        + col_block0.astype(jnp.uint32)
    )

    vrun = jnp.full(cshape, -jnp.inf, jnp.float32)
    irun = jnp.zeros(cshape, jnp.int32)

    for k in range(CHUNKS_PER_BLOCK):
        x1 = base_vec + (group_scalar + _u32(k * CHUNK))
        bits = _threefry2x32_zero_hi(x1)
        fb = (bits >> _u32(9)) | _u32(0x3F800000)
        f = pltpu.bitcast(fb, jnp.float32) - jnp.float32(1.0)
        g = -jnp.log(-jnp.log(jnp.maximum(f, jnp.float32(_TINY))))
        lch = logits_ref[:, k * CHUNK : (k + 1) * CHUNK]
        col = lane_i32 + (col_block0 + k * CHUNK)
        v = jnp.where(col < VOCAB, lch + g, -jnp.inf)
        take = v > vrun
        vrun = jnp.maximum(vrun, v)
        irun = jnp.where(take, col, irun)

    gmax = jnp.max(vrun, axis=1, keepdims=True)
    cand = jnp.where(vrun == gmax, irun, jnp.int32(0x7FFFFFFF))
    gidx = jnp.min(cand, axis=1, keepdims=True)

    @pl.when(j == 0)
    def _init():
        best_val[...] = jnp.full((ROWS_PER_GROUP, 1), -jnp.inf, jnp.float32)
        best_idx[...] = jnp.zeros((ROWS_PER_GROUP, 1), jnp.int32)

    prev_v = best_val[...]
    prev_i = best_idx[...]
    better = gmax > prev_v
    best_val[...] = jnp.where(better, gmax, prev_v)
    best_idx[...] = jnp.where(better, gidx, prev_i)

    @pl.when(j == NUM_BLOCKS - 1)
    def _done():
        out_ref[...] = best_idx[...]


@functools.partial(jax.jit, static_argnames=())
def kernel(logits):
    out = pl.pallas_call(
        _sample_block,
        grid=(NUM_GROUPS, NUM_BLOCKS),
        in_specs=[
            pl.BlockSpec((ROWS_PER_GROUP, BLOCK_W), lambda s, j: (s, j)),
        ],
        out_specs=pl.BlockSpec((ROWS_PER_GROUP, 1), lambda s, j: (s, 0)),
        out_shape=jax.ShapeDtypeStruct((BATCH, 1), jnp.int32),
        scratch_shapes=[
            pltpu.VMEM((ROWS_PER_GROUP, 1), jnp.float32),
            pltpu.VMEM((ROWS_PER_GROUP, 1), jnp.int32),
        ],
        compiler_params=pltpu.CompilerParams(
            dimension_semantics=("arbitrary", "arbitrary"),
        ),
    )(logits)
    return out.reshape(BATCH)


# speedup vs baseline: 2.4964x; 1.0229x over previous
import functools

import jax
import jax.numpy as jnp
from jax.experimental import pallas as pl
from jax.experimental.pallas import tpu as pltpu

BATCH = 128
VOCAB = 100000
BLOCK_W = 25600
NUM_BLOCKS = (VOCAB + BLOCK_W - 1) // BLOCK_W
CHUNK = 256
CHUNKS_PER_BLOCK = BLOCK_W // CHUNK
ROWS_PER_GROUP = 8
NUM_GROUPS = BATCH // ROWS_PER_GROUP

_KEY0 = 928981903
_KEY1 = 3453687069
_KS2 = _KEY0 ^ _KEY1 ^ 0x1BD11BDA


def _u32(x):
    return jnp.uint32(x)


def _rotl(x, d):
    return (x << _u32(d)) | (x >> _u32(32 - d))


def _threefry2x32_zero_hi(x1):
    ks = (_KEY0, _KEY1, _KS2)
    rot0 = (13, 15, 26, 6)
    rot1 = (17, 29, 16, 24)
    x0 = x1 + _u32(_KEY0)
    x1 = _rotl(x1, rot0[0]) ^ x0
    first = True
    for rots, ka, kb, inc in (
        (rot0, 1, 2, 1),
        (rot1, 2, 0, 2),
        (rot0, 0, 1, 3),
        (rot1, 1, 2, 4),
        (rot0, 2, 0, 5),
    ):
        for r in rots:
            if first:
                first = False
                continue
            x0 = x0 + x1
            x1 = _rotl(x1, r)
            x1 = x1 ^ x0
        x0 = x0 + _u32(ks[ka])
        x1 = x1 + _u32((ks[kb] + inc) & 0xFFFFFFFF)
    return x0 ^ x1


_TINY = float(jnp.finfo(jnp.float32).tiny)


def _sample_block(logits_ref, out_ref, best_val, best_idx):
    s = pl.program_id(0)
    j = pl.program_id(1)
    cshape = (ROWS_PER_GROUP, CHUNK)

    row_off = jax.lax.broadcasted_iota(jnp.uint32, cshape, 0) * _u32(VOCAB)
    lane = jax.lax.broadcasted_iota(jnp.uint32, cshape, 1)
    base_vec = row_off + lane
    lane_i32 = lane.astype(jnp.int32)

    col_block0 = j * BLOCK_W
    group_scalar = (
        s.astype(jnp.uint32) * _u32(ROWS_PER_GROUP * VOCAB)
        + _u32(_KEY1)
        + col_block0.astype(jnp.uint32)
    )

    @pl.when(j == 0)
    def _init():
        best_val[...] = jnp.full(cshape, -jnp.inf, jnp.float32)
        best_idx[...] = jnp.zeros(cshape, jnp.int32)

    vrun = best_val[...]
    irun = best_idx[...]

    for k in range(CHUNKS_PER_BLOCK):
        x1 = base_vec + (group_scalar + _u32(k * CHUNK))
        bits = _threefry2x32_zero_hi(x1)
        fb = (bits >> _u32(9)) | _u32(0x3F800000)
        f = pltpu.bitcast(fb, jnp.float32) - jnp.float32(1.0)
        g = -jnp.log(-jnp.log(jnp.maximum(f, jnp.float32(_TINY))))
        lch = logits_ref[:, k * CHUNK : (k + 1) * CHUNK]
        col = lane_i32 + (col_block0 + k * CHUNK)
        v = jnp.where(col < VOCAB, lch + g, -jnp.inf)
        take = v > vrun
        vrun = jnp.maximum(vrun, v)
        irun = jnp.where(take, col, irun)

    best_val[...] = vrun
    best_idx[...] = irun

    @pl.when(j == NUM_BLOCKS - 1)
    def _done():
        gmax = jnp.max(vrun, axis=1, keepdims=True)
        cand = jnp.where(vrun == gmax, irun, jnp.int32(0x7FFFFFFF))
        out_ref[...] = jnp.min(cand, axis=1, keepdims=True)


@functools.partial(jax.jit, static_argnames=())
def kernel(logits):
    out = pl.pallas_call(
        _sample_block,
        grid=(NUM_GROUPS, NUM_BLOCKS),
        in_specs=[
            pl.BlockSpec((ROWS_PER_GROUP, BLOCK_W), lambda s, j: (s, j)),
        ],
        out_specs=pl.BlockSpec((ROWS_PER_GROUP, 1), lambda s, j: (s, 0)),
        out_shape=jax.ShapeDtypeStruct((BATCH, 1), jnp.int32),
        scratch_shapes=[
            pltpu.VMEM((ROWS_PER_GROUP, CHUNK), jnp.float32),
            pltpu.VMEM((ROWS_PER_GROUP, CHUNK), jnp.int32),
        ],
        compiler_params=pltpu.CompilerParams(
            dimension_semantics=("arbitrary", "arbitrary"),
        ),
    )(logits)
    return out.reshape(BATCH)


# speedup vs baseline: 2.5861x; 1.0359x over previous
import functools

import jax
import jax.numpy as jnp
from jax.experimental import pallas as pl
from jax.experimental.pallas import tpu as pltpu

BATCH = 128
VOCAB = 100000
CHUNK = 256
ROWS_PER_GROUP = 8
NUM_GROUPS = BATCH // ROWS_PER_GROUP

_STARTS = list(range(0, VOCAB - CHUNK + 1, CHUNK))
if _STARTS[-1] + CHUNK < VOCAB:
    _STARTS.append(VOCAB - CHUNK)

_KEY0 = 928981903
_KEY1 = 3453687069
_KS2 = _KEY0 ^ _KEY1 ^ 0x1BD11BDA


def _u32(x):
    return jnp.uint32(x)


def _rotl(x, d):
    return (x << _u32(d)) | (x >> _u32(32 - d))


def _threefry2x32_zero_hi(x1):
    ks = (_KEY0, _KEY1, _KS2)
    rot0 = (13, 15, 26, 6)
    rot1 = (17, 29, 16, 24)
    x0 = x1 + _u32(_KEY0)
    x1 = _rotl(x1, rot0[0]) ^ x0
    first = True
    for rots, ka, kb, inc in (
        (rot0, 1, 2, 1),
        (rot1, 2, 0, 2),
        (rot0, 0, 1, 3),
        (rot1, 1, 2, 4),
        (rot0, 2, 0, 5),
    ):
        for r in rots:
            if first:
                first = False
                continue
            x0 = x0 + x1
            x1 = _rotl(x1, r)
            x1 = x1 ^ x0
        x0 = x0 + _u32(ks[ka])
        x1 = x1 + _u32((ks[kb] + inc) & 0xFFFFFFFF)
    return x0 ^ x1


_TINY = float(jnp.finfo(jnp.float32).tiny)


def _sample_block(logits_ref, out_ref):
    s = pl.program_id(0)
    cshape = (ROWS_PER_GROUP, CHUNK)

    row_off = jax.lax.broadcasted_iota(jnp.uint32, cshape, 0) * _u32(VOCAB)
    lane = jax.lax.broadcasted_iota(jnp.uint32, cshape, 1)
    base_vec = row_off + lane
    lane_i32 = lane.astype(jnp.int32)

    group_scalar = s.astype(jnp.uint32) * _u32(ROWS_PER_GROUP * VOCAB) + _u32(
        _KEY1
    )

    vrun = jnp.full(cshape, -jnp.inf, jnp.float32)
    irun = jnp.zeros(cshape, jnp.int32)

    for c0 in _STARTS:
        x1 = base_vec + (group_scalar + _u32(c0))
        bits = _threefry2x32_zero_hi(x1)
        fb = (bits >> _u32(9)) | _u32(0x3F800000)
        f = pltpu.bitcast(fb, jnp.float32) - jnp.float32(1.0)
        g = -jnp.log(-jnp.log(jnp.maximum(f, jnp.float32(_TINY))))
        v = logits_ref[:, c0 : c0 + CHUNK] + g
        col = lane_i32 + c0
        take = v > vrun
        vrun = jnp.maximum(vrun, v)
        irun = jnp.where(take, col, irun)

    gmax = jnp.max(vrun, axis=1, keepdims=True)
    cand = jnp.where(vrun == gmax, irun, jnp.int32(0x7FFFFFFF))
    out_ref[...] = jnp.min(cand, axis=1, keepdims=True)


@functools.partial(jax.jit, static_argnames=())
def kernel(logits):
    out = pl.pallas_call(
        _sample_block,
        grid=(NUM_GROUPS,),
        in_specs=[
            pl.BlockSpec((ROWS_PER_GROUP, VOCAB), lambda s: (s, 0)),
        ],
        out_specs=pl.BlockSpec((ROWS_PER_GROUP, 1), lambda s: (s, 0)),
        out_shape=jax.ShapeDtypeStruct((BATCH, 1), jnp.int32),
        compiler_params=pltpu.CompilerParams(
            dimension_semantics=("arbitrary",),
        ),
    )(logits)
    return out.reshape(BATCH)


# speedup vs baseline: 2.5936x; 1.0029x over previous
import functools

import jax
import jax.numpy as jnp
from jax.experimental import pallas as pl
from jax.experimental.pallas import tpu as pltpu

BATCH = 128
VOCAB = 100000
CHUNK = 1024
ROWS_PER_GROUP = 8
NUM_GROUPS = BATCH // ROWS_PER_GROUP

_STARTS = list(range(0, VOCAB - CHUNK + 1, CHUNK))
if _STARTS[-1] + CHUNK < VOCAB:
    _STARTS.append(VOCAB - CHUNK)

_KEY0 = 928981903
_KEY1 = 3453687069
_KS2 = _KEY0 ^ _KEY1 ^ 0x1BD11BDA


def _u32(x):
    return jnp.uint32(x)


def _rotl(x, d):
    return (x << _u32(d)) | (x >> _u32(32 - d))


def _threefry2x32_zero_hi(x1):
    ks = (_KEY0, _KEY1, _KS2)
    rot0 = (13, 15, 26, 6)
    rot1 = (17, 29, 16, 24)
    x0 = x1 + _u32(_KEY0)
    x1 = _rotl(x1, rot0[0]) ^ x0
    first = True
    for rots, ka, kb, inc in (
        (rot0, 1, 2, 1),
        (rot1, 2, 0, 2),
        (rot0, 0, 1, 3),
        (rot1, 1, 2, 4),
        (rot0, 2, 0, 5),
    ):
        for r in rots:
            if first:
                first = False
                continue
            x0 = x0 + x1
            x1 = _rotl(x1, r)
            x1 = x1 ^ x0
        x0 = x0 + _u32(ks[ka])
        x1 = x1 + _u32((ks[kb] + inc) & 0xFFFFFFFF)
    return x0 ^ x1


_TINY = float(jnp.finfo(jnp.float32).tiny)


def _sample_block(logits_ref, out_ref):
    s = pl.program_id(0)
    cshape = (ROWS_PER_GROUP, CHUNK)

    row_off = jax.lax.broadcasted_iota(jnp.uint32, cshape, 0) * _u32(VOCAB)
    lane = jax.lax.broadcasted_iota(jnp.uint32, cshape, 1)
    base_vec = row_off + lane
    lane_i32 = lane.astype(jnp.int32)

    group_scalar = s.astype(jnp.uint32) * _u32(ROWS_PER_GROUP * VOCAB) + _u32(
        _KEY1
    )

    vrun = jnp.full(cshape, -jnp.inf, jnp.float32)
    irun = jnp.zeros(cshape, jnp.int32)

    for c0 in _STARTS:
        x1 = base_vec + (group_scalar + _u32(c0))
        bits = _threefry2x32_zero_hi(x1)
        fb = (bits >> _u32(9)) | _u32(0x3F800000)
        f = pltpu.bitcast(fb, jnp.float32) - jnp.float32(1.0)
        g = -jnp.log(-jnp.log(jnp.maximum(f, jnp.float32(_TINY))))
        v = logits_ref[:, c0 : c0 + CHUNK] + g
        col = lane_i32 + c0
        take = v > vrun
        vrun = jnp.maximum(vrun, v)
        irun = jnp.where(take, col, irun)

    gmax = jnp.max(vrun, axis=1, keepdims=True)
    cand = jnp.where(vrun == gmax, irun, jnp.int32(0x7FFFFFFF))
    out_ref[...] = jnp.min(cand, axis=1, keepdims=True)


@functools.partial(jax.jit, static_argnames=())
def kernel(logits):
    out = pl.pallas_call(
        _sample_block,
        grid=(NUM_GROUPS,),
        in_specs=[
            pl.BlockSpec((ROWS_PER_GROUP, VOCAB), lambda s: (s, 0)),
        ],
        out_specs=pl.BlockSpec((ROWS_PER_GROUP, 1), lambda s: (s, 0)),
        out_shape=jax.ShapeDtypeStruct((BATCH, 1), jnp.int32),
        compiler_params=pltpu.CompilerParams(
            dimension_semantics=("arbitrary",),
        ),
    )(logits)
    return out.reshape(BATCH)
